# baseline (speedup 1.0000x reference)
import functools

import jax
import jax.numpy as jnp
from jax import lax
from jax.experimental import pallas as pl
from jax.experimental.pallas import tpu as pltpu
from jax.experimental.pallas import tpu_sc as plsc

_NC = 2
_NS = 16
_NW = _NC * _NS

_B = 16384
_D = 64
_BPW = _B // _NW
_CH = 128
_NCH = _BPW // _CH


def _embed_body(idx_hbm, table_hbm, out_hbm, idx_v, rows_v, sem):
    wid = lax.axis_index("s") * _NC + lax.axis_index("c")
    pltpu.sync_copy(idx_hbm.at[pl.ds(wid * _NCH, _NCH)], idx_v)
    copies = [
        pltpu.async_copy(
            table_hbm.at[idx_v.at[t]],
            rows_v.at[pl.ds(t * _CH, _CH)],
            sem,
        )
        for t in range(_NCH)
    ]
    for c in copies:
        c.wait()
    pltpu.sync_copy(rows_v, out_hbm.at[pl.ds(wid * _BPW, _BPW)])


@jax.jit
def _embed(idx, table):
    run = pl.kernel(
        _embed_body,
        out_type=jax.ShapeDtypeStruct((_B, _D), jnp.float32),
        mesh=plsc.VectorSubcoreMesh(
            core_axis_name="c", subcore_axis_name="s",
            num_cores=_NC, num_subcores=_NS,
        ),
        scratch_types=[
            pltpu.VMEM((_NCH, _CH), jnp.int32),
            pltpu.VMEM((_BPW, _D), jnp.float32),
            pltpu.SemaphoreType.DMA,
        ],
        compiler_params=pltpu.CompilerParams(use_tc_tiling_on_sc=False),
    )
    return run(idx, table)


def kernel(x, table):
    idx = x.astype(jnp.int32).reshape(_NW * _NCH, _CH)
    return _embed(idx, table)


# speedup vs baseline: 1.0853x; 1.0853x over previous
import functools

import jax
import jax.numpy as jnp
from jax import lax
from jax.experimental import pallas as pl
from jax.experimental.pallas import tpu as pltpu
from jax.experimental.pallas import tpu_sc as plsc

_NC = 2
_NS = 16
_NW = _NC * _NS

_B = 16384
_D = 64
_BPW = _B // _NW
_CH = 128
_NCH = _BPW // _CH


def _embed_body(idx_hbm, table_hbm, out_hbm, idx_v, rows_v, sem):
    wid = lax.axis_index("s") * _NC + lax.axis_index("c")
    pltpu.sync_copy(idx_hbm.at[pl.ds(wid * _NCH, _NCH)], idx_v)
    copies = [
        pltpu.async_copy(
            table_hbm.at[idx_v.at[t]],
            rows_v.at[pl.ds(t * _CH, _CH)],
            sem,
        )
        for t in range(_NCH)
    ]
    for c in copies:
        c.wait()
    pltpu.sync_copy(rows_v, out_hbm.at[pl.ds(wid * _BPW, _BPW), pl.ds(0, _D)])


@jax.jit
def _embed(idx, table):
    run = pl.kernel(
        _embed_body,
        out_type=jax.ShapeDtypeStruct((_B, 128), jnp.float32),
        mesh=plsc.VectorSubcoreMesh(
            core_axis_name="c", subcore_axis_name="s",
            num_cores=_NC, num_subcores=_NS,
        ),
        scratch_types=[
            pltpu.VMEM((_NCH, _CH), jnp.int32),
            pltpu.VMEM((_BPW, _D), jnp.float32),
            pltpu.SemaphoreType.DMA,
        ],
        compiler_params=pltpu.CompilerParams(use_tc_tiling_on_sc=False),
    )
    return run(idx, table)


def kernel(x, table):
    idx = x.astype(jnp.int32).reshape(_NW * _NCH, _CH)
    out128 = _embed(idx, table)
    return jax.lax.slice(out128, (0, 0), (_B, _D))


# speedup vs baseline: 1.8861x; 1.7379x over previous
import functools

import jax
import jax.numpy as jnp
from jax import lax
from jax.experimental import pallas as pl
from jax.experimental.pallas import tpu as pltpu
from jax.experimental.pallas import tpu_sc as plsc

_NC = 2
_NS = 16
_NW = _NC * _NS

_B = 16384
_D = 64
_ROWS_PER_TILE = _D // _NW
_V = 100001
_OUT_CHUNK = 8192


def _embed_body(idx_hbm, tab_hbm, out_hbm, idx_v, row_v, stage_v):
    wid = lax.axis_index("s") * _NC + lax.axis_index("c")
    pltpu.sync_copy(idx_hbm, idx_v)
    zeros16 = jnp.zeros((16,), jnp.int32)
    for p in range(_ROWS_PER_TILE):
        d = wid + _NW * p
        pltpu.sync_copy(tab_hbm.at[pl.ds(d, 1)], row_v)
        for h in range(_B // _OUT_CHUNK):

            def step(i, _):
                idx16 = idx_v[pl.ds(h * _OUT_CHUNK + i * 16, 16)]
                vals = plsc.load_gather(row_v, [zeros16, idx16])
                stage_v[pl.ds(i * 16, 16)] = vals
                return ()

            lax.fori_loop(0, _OUT_CHUNK // 16, step, (), unroll=4)
            pltpu.sync_copy(
                stage_v,
                out_hbm.at[d, pl.ds(h * _OUT_CHUNK, _OUT_CHUNK)],
            )


@jax.jit
def _embed(idx, tab_t):
    run = pl.kernel(
        _embed_body,
        out_type=jax.ShapeDtypeStruct((_D, _B), jnp.float32),
        mesh=plsc.VectorSubcoreMesh(
            core_axis_name="c", subcore_axis_name="s",
            num_cores=_NC, num_subcores=_NS,
        ),
        scratch_types=[
            pltpu.VMEM((_B,), jnp.int32),
            pltpu.VMEM((1, _V), jnp.float32),
            pltpu.VMEM((_OUT_CHUNK,), jnp.float32),
        ],
        compiler_params=pltpu.CompilerParams(needs_layout_passes=False),
    )
    return run(idx, tab_t)


def kernel(x, table):
    out_t = _embed(x.astype(jnp.int32), table.T)
    return out_t.T


# speedup vs baseline: 2.9299x; 1.5534x over previous
import functools

import jax
import jax.numpy as jnp
from jax import lax
from jax.experimental import pallas as pl
from jax.experimental.pallas import tpu as pltpu
from jax.experimental.pallas import tpu_sc as plsc

_NC = 2
_NS = 16
_NW = _NC * _NS

_B = 16384
_D = 64
_ROWS_PER_TILE = _D // _NW
_V = 100001
_OUT_CHUNK = 4096


def _embed_body(idx_hbm, tab_hbm, out_hbm, idx_v, row_v, stage_v, sem):
    wid = lax.axis_index("s") * _NC + lax.axis_index("c")
    pltpu.sync_copy(idx_hbm, idx_v)
    zeros16 = jnp.zeros((16,), jnp.int32)
    pending = [None, None]
    for p in range(_ROWS_PER_TILE):
        d = wid + _NW * p
        pltpu.sync_copy(tab_hbm.at[pl.ds(d, 1)], row_v)
        for h in range(_B // _OUT_CHUNK):
            b = (p * (_B // _OUT_CHUNK) + h) % 2
            if pending[b] is not None:
                pending[b].wait()

            @functools.partial(
                plsc.parallel_loop, 0, _OUT_CHUNK // 16, unroll=8
            )
            def step(i, _h=h, _b=b):
                idx16 = idx_v[pl.ds(_h * _OUT_CHUNK + i * 16, 16)]
                vals = plsc.load_gather(row_v, [zeros16, idx16])
                stage_v[pl.ds(_b * _OUT_CHUNK + i * 16, 16)] = vals

            pending[b] = pltpu.async_copy(
                stage_v.at[pl.ds(b * _OUT_CHUNK, _OUT_CHUNK)],
                out_hbm.at[d, pl.ds(h * _OUT_CHUNK, _OUT_CHUNK)],
                sem,
            )
    for c in pending:
        if c is not None:
            c.wait()


@jax.jit
def _embed(idx, tab_t):
    run = pl.kernel(
        _embed_body,
        out_type=jax.ShapeDtypeStruct((_D, _B), jnp.float32),
        mesh=plsc.VectorSubcoreMesh(
            core_axis_name="c", subcore_axis_name="s",
            num_cores=_NC, num_subcores=_NS,
        ),
        scratch_types=[
            pltpu.VMEM((_B,), jnp.int32),
            pltpu.VMEM((1, _V), jnp.float32),
            pltpu.VMEM((2 * _OUT_CHUNK,), jnp.float32),
            pltpu.SemaphoreType.DMA,
        ],
        compiler_params=pltpu.CompilerParams(needs_layout_passes=False),
    )
    return run(idx, tab_t)


def kernel(x, table):
    out_t = _embed(x.astype(jnp.int32), table.T)
    return out_t.T
